# baseline (speedup 1.0000x reference)
import functools

import jax
import jax.numpy as jnp
from jax import lax
from jax.experimental import pallas as pl
from jax.experimental.pallas import tpu as pltpu
from jax.experimental.pallas import tpu_sc as plsc

BATCH = 16384
FACTOR = 64
NW = 32
B_PER_W = BATCH // NW
NROWS = 1000000
NWIN = 1954
WROWS = 512
LAST_WSTART = 999552
STAGE_ROWS = BATCH + 128
LISTCAP = 2048
WIDE = 128

_mesh = plsc.VectorSubcoreMesh(core_axis_name="c", subcore_axis_name="s")
_params = pltpu.CompilerParams(needs_layout_passes=False, use_tc_tiling_on_sc=True)


@functools.partial(
    pl.kernel,
    mesh=_mesh,
    out_type=(
        jax.ShapeDtypeStruct((STAGE_ROWS, WIDE), jnp.float32),
        jax.ShapeDtypeStruct((STAGE_ROWS, WIDE), jnp.float32),
    ),
    scratch_types=[
        pltpu.VMEM((BATCH,), jnp.int32),
        pltpu.VMEM((LISTCAP + 16,), jnp.int32),
        pltpu.VMEM((LISTCAP + 16,), jnp.int32),
        pltpu.VMEM((32,), jnp.int32),
        pltpu.VMEM((32,), jnp.int32),
        pltpu.VMEM((8, 8, WROWS), jnp.float32),
        pltpu.VMEM((8, 8, WROWS), jnp.float32),
        pltpu.VMEM((128, WIDE), jnp.float32),
        pltpu.VMEM((128,), jnp.int32),
        pltpu.SemaphoreType.DMA,
        pltpu.SemaphoreType.DMA,
        pltpu.SemaphoreType.DMA,
    ],
    compiler_params=_params,
)
def _k1(user_hbm, item_hbm, eu_hbm, ei_hbm, su_hbm, si_hbm,
        idx_v, rl, bl, rblk, bblk, blka, blkb, rowbuf, bchunk,
        sema, semb, sems):
    wid = lax.axis_index("s") * 2 + lax.axis_index("c")
    lo_w = (wid * NWIN) // NW
    hi_w = ((wid + 1) * NWIN) // NW
    lane = lax.iota(jnp.int32, 16)
    ci = lane & 7
    cbs = [2 * k + (lane >> 3) for k in range(4)]

    def reset_bchunk():
        for k in range(8):
            bchunk[pl.ds(k * 16, 16)] = BATCH + k * 16 + lane

    reset_bchunk()

    def flush(stage_hbm):
        pltpu.async_copy(rowbuf, stage_hbm.at[bchunk], sems).wait()
        reset_bchunk()

    def wstart(j):
        return pl.multiple_of(jnp.minimum(j * WROWS, LAST_WSTART), 128)

    def blk_slice(tref, j):
        return tref.at[:, :, pl.ds(wstart(j), WROWS)]

    def run_table(idx_hbm, tref, stage_hbm):
        pltpu.sync_copy(idx_hbm, idx_v)

        def scan_body(v, pos):
            r16 = idx_v[pl.ds(v * 16, 16)]
            w = r16 >> 9
            m = (w >= lo_w) & (w < hi_w)
            plsc.store_compressed(rl.at[pl.ds(pos, 16)], r16, mask=m)
            plsc.store_compressed(bl.at[pl.ds(pos, 16)], v * 16 + lane, mask=m)
            return jnp.minimum(pos + jnp.sum(m.astype(jnp.int32)), LISTCAP)

        cnt = lax.fori_loop(0, BATCH // 16, scan_body, 0)
        rl[pl.ds(cnt, 16)] = jnp.full((16,), -1, jnp.int32)

        nv = (cnt + 15) >> 4

        def process(j, blkref, hc):
            start = wstart(j)

            def sub(v, hc):
                r16 = rl[pl.ds(v * 16, 16)]
                m = (r16 >> 9) == j
                pc = jnp.sum(m.astype(jnp.int32))

                def have(hc):
                    plsc.store_compressed(rblk.at[pl.ds(0, 16)], r16, mask=m)
                    plsc.store_compressed(
                        bblk.at[pl.ds(0, 16)], bl[pl.ds(v * 16, 16)], mask=m)

                    def per_hit(h, hc):
                        rvec = rblk[pl.ds(h, 16)]
                        bvec = bblk[pl.ds(h, 16)]
                        ri = jnp.full((16,), rvec[0] - start, jnp.int32)
                        for k in range(4):
                            val = plsc.load_gather(blkref, [cbs[k], ci, ri])
                            rowbuf[hc, pl.ds(k * 16, 16)] = val
                        grp = (hc >> 4) * 16
                        off = hc & 15
                        cur = bchunk[pl.ds(grp, 16)]
                        bchunk[pl.ds(grp, 16)] = jnp.where(
                            lane == off, jnp.full((16,), bvec[0], jnp.int32), cur)
                        hc = hc + 1

                        def do_flush(hc):
                            flush(stage_hbm)
                            return 0

                        return lax.cond(hc == 128, do_flush, lambda hc: hc, hc)

                    return lax.fori_loop(0, pc, per_hit, hc)

                return lax.cond(pc > 0, have, lambda hc: hc, hc)

            return lax.fori_loop(0, nv, sub, hc)

        def fire(j, buf, sem):
            pltpu.async_copy(blk_slice(tref, j), buf, sem)

        def wait(j, buf, sem):
            pltpu.make_async_copy(blk_slice(tref, j), buf, sem).wait()

        @pl.when(lo_w < hi_w)
        def _():
            fire(lo_w, blka, sema)

        npairs = (hi_w - lo_w + 1) // 2

        def pair(t, hc):
            j0 = lo_w + 2 * t
            j1 = j0 + 1
            j2 = j0 + 2

            @pl.when(j1 < hi_w)
            def _():
                fire(j1, blkb, semb)

            wait(j0, blka, sema)
            hc = process(j0, blka, hc)

            @pl.when(j2 < hi_w)
            def _():
                fire(j2, blka, sema)

            def do_b(hc):
                wait(j1, blkb, semb)
                return process(j1, blkb, hc)

            return lax.cond(j1 < hi_w, do_b, lambda hc: hc, hc)

        hc = lax.fori_loop(0, npairs, pair, 0)

        @pl.when(hc > 0)
        def _():
            flush(stage_hbm)

    run_table(user_hbm, eu_hbm, su_hbm)
    run_table(item_hbm, ei_hbm, si_hbm)


@functools.partial(
    pl.kernel,
    mesh=_mesh,
    out_type=jax.ShapeDtypeStruct((BATCH,), jnp.float32),
    scratch_types=[
        pltpu.VMEM((128, WIDE), jnp.float32),
        pltpu.VMEM((128, WIDE), jnp.float32),
        pltpu.VMEM((B_PER_W,), jnp.float32),
    ],
    compiler_params=_params,
)
def _k2(su_hbm, si_hbm, out_hbm, ubuf, ibuf, out_v):
    wid = lax.axis_index("s") * 2 + lax.axis_index("c")
    base = wid * B_PER_W
    lane = lax.iota(jnp.int32, 16)

    for j in range(4):
        pltpu.sync_copy(su_hbm.at[pl.ds(base + j * 128, 128), :], ubuf)
        pltpu.sync_copy(si_hbm.at[pl.ds(base + j * 128, 128), :], ibuf)

        def body(g, carry):
            row = g * 16 + lane
            acc = jnp.zeros((16,), jnp.float32)
            for c in range(FACTOR):
                col = jnp.full((16,), c, jnp.int32)
                u = plsc.load_gather(ubuf, [row, col])
                v = plsc.load_gather(ibuf, [row, col])
                acc = acc + u * v
            out_v[pl.ds(j * 128 + g * 16, 16)] = acc
            return carry

        lax.fori_loop(0, 8, body, 0)

    pltpu.sync_copy(out_v, out_hbm.at[pl.ds(base, B_PER_W)])


def kernel(user, item, embed_user, embed_item):
    eu3 = embed_user.T.reshape(8, 8, NROWS)
    ei3 = embed_item.T.reshape(8, 8, NROWS)
    su, si = _k1(user.astype(jnp.int32), item.astype(jnp.int32), eu3, ei3)
    return _k2(su, si)


# speedup vs baseline: 1.0484x; 1.0484x over previous
import functools

import jax
import jax.numpy as jnp
from jax import lax
from jax.experimental import pallas as pl
from jax.experimental.pallas import tpu as pltpu
from jax.experimental.pallas import tpu_sc as plsc

BATCH = 16384
FACTOR = 64
NW = 32
B_PER_W = BATCH // NW
NROWS = 1000000
NWIN = 1954
WROWS = 512
LAST_WSTART = 999552
STAGE_ROWS = BATCH + 128
LISTCAP = 2048
WIDE = 128

_mesh = plsc.VectorSubcoreMesh(core_axis_name="c", subcore_axis_name="s")
_params = pltpu.CompilerParams(needs_layout_passes=False, use_tc_tiling_on_sc=True)


@functools.partial(
    pl.kernel,
    mesh=_mesh,
    out_type=(
        jax.ShapeDtypeStruct((STAGE_ROWS, WIDE), jnp.float32),
        jax.ShapeDtypeStruct((STAGE_ROWS, WIDE), jnp.float32),
    ),
    scratch_types=[
        pltpu.VMEM((BATCH,), jnp.int32),
        pltpu.VMEM((LISTCAP + 64,), jnp.int32),
        pltpu.VMEM((LISTCAP + 64,), jnp.int32),
        pltpu.VMEM((32,), jnp.int32),
        pltpu.VMEM((32,), jnp.int32),
        pltpu.VMEM((8, 8, WROWS), jnp.float32),
        pltpu.VMEM((8, 8, WROWS), jnp.float32),
        pltpu.VMEM((128, WIDE), jnp.float32),
        pltpu.VMEM((128,), jnp.int32),
        pltpu.SemaphoreType.DMA,
        pltpu.SemaphoreType.DMA,
        pltpu.SemaphoreType.DMA,
    ],
    compiler_params=_params,
)
def _k1(user_hbm, item_hbm, eu_hbm, ei_hbm, su_hbm, si_hbm,
        idx_v, rl, bl, rblk, bblk, blka, blkb, rowbuf, bchunk,
        sema, semb, sems):
    wid = lax.axis_index("s") * 2 + lax.axis_index("c")
    lo_w = (wid * NWIN) // NW
    hi_w = ((wid + 1) * NWIN) // NW
    lane = lax.iota(jnp.int32, 16)
    ci = lane & 7
    cbs = [2 * k + (lane >> 3) for k in range(4)]

    def reset_bchunk():
        for k in range(8):
            bchunk[pl.ds(k * 16, 16)] = BATCH + k * 16 + lane

    reset_bchunk()

    def flush(stage_hbm):
        pltpu.async_copy(rowbuf, stage_hbm.at[bchunk], sems).wait()
        reset_bchunk()

    def wstart(j):
        return pl.multiple_of(jnp.minimum(j * WROWS, LAST_WSTART), 128)

    def blk_slice(tref, j):
        return tref.at[:, :, pl.ds(wstart(j), WROWS)]

    def run_table(idx_hbm, tref, stage_hbm):
        pltpu.sync_copy(idx_hbm, idx_v)

        def scan_body(v, pos):
            r16 = idx_v[pl.ds(v * 16, 16)]
            w = r16 >> 9
            m = (w >= lo_w) & (w < hi_w)
            plsc.store_compressed(rl.at[pl.ds(pos, 16)], r16, mask=m)
            plsc.store_compressed(bl.at[pl.ds(pos, 16)], v * 16 + lane, mask=m)
            return jnp.minimum(pos + jnp.sum(m.astype(jnp.int32)), LISTCAP)

        cnt = lax.fori_loop(0, BATCH // 16, scan_body, 0)
        for k in range(4):
            rl[pl.ds(cnt + k * 16, 16)] = jnp.full((16,), -1, jnp.int32)

        nq = (cnt + 63) >> 6

        def process(j, blkref, hc):
            start = wstart(j)

            def sub(q, hc):
                r16s, ms, pcs = [], [], []
                for k in range(4):
                    r16 = rl[pl.ds(q * 64 + k * 16, 16)]
                    m = (r16 >> 9) == j
                    r16s.append(r16)
                    ms.append(m)
                    pcs.append(jnp.sum(m.astype(jnp.int32)))

                for k in range(4):
                    r16, m, pc = r16s[k], ms[k], pcs[k]

                    def have(hc, r16=r16, m=m, pc=pc, k=k):
                        plsc.store_compressed(rblk.at[pl.ds(0, 16)], r16, mask=m)
                        plsc.store_compressed(
                            bblk.at[pl.ds(0, 16)],
                            bl[pl.ds(q * 64 + k * 16, 16)], mask=m)

                        def per_hit(h, hc):
                            rvec = rblk[pl.ds(h, 16)]
                            bvec = bblk[pl.ds(h, 16)]
                            ri = jnp.full((16,), rvec[0] - start, jnp.int32)
                            for t in range(4):
                                val = plsc.load_gather(blkref, [cbs[t], ci, ri])
                                rowbuf[hc, pl.ds(t * 16, 16)] = val
                            grp = (hc >> 4) * 16
                            off = hc & 15
                            cur = bchunk[pl.ds(grp, 16)]
                            bchunk[pl.ds(grp, 16)] = jnp.where(
                                lane == off,
                                jnp.full((16,), bvec[0], jnp.int32), cur)
                            hc = hc + 1

                            def do_flush(hc):
                                flush(stage_hbm)
                                return 0

                            return lax.cond(hc == 128, do_flush,
                                            lambda hc: hc, hc)

                        return lax.fori_loop(0, pc, per_hit, hc)

                    hc = lax.cond(pc > 0, have, lambda hc: hc, hc)
                return hc

            return lax.fori_loop(0, nq, sub, hc)

        def fire(j, buf, sem):
            pltpu.async_copy(blk_slice(tref, j), buf, sem)

        def wait(j, buf, sem):
            pltpu.make_async_copy(blk_slice(tref, j), buf, sem).wait()

        @pl.when(lo_w < hi_w)
        def _():
            fire(lo_w, blka, sema)

        npairs = (hi_w - lo_w + 1) // 2

        def pair(t, hc):
            j0 = lo_w + 2 * t
            j1 = j0 + 1
            j2 = j0 + 2

            @pl.when(j1 < hi_w)
            def _():
                fire(j1, blkb, semb)

            wait(j0, blka, sema)
            hc = process(j0, blka, hc)

            @pl.when(j2 < hi_w)
            def _():
                fire(j2, blka, sema)

            def do_b(hc):
                wait(j1, blkb, semb)
                return process(j1, blkb, hc)

            return lax.cond(j1 < hi_w, do_b, lambda hc: hc, hc)

        hc = lax.fori_loop(0, npairs, pair, 0)

        @pl.when(hc > 0)
        def _():
            flush(stage_hbm)

    run_table(user_hbm, eu_hbm, su_hbm)
    run_table(item_hbm, ei_hbm, si_hbm)


@functools.partial(
    pl.kernel,
    mesh=_mesh,
    out_type=jax.ShapeDtypeStruct((BATCH,), jnp.float32),
    scratch_types=[
        pltpu.VMEM((128, WIDE), jnp.float32),
        pltpu.VMEM((128, WIDE), jnp.float32),
        pltpu.VMEM((128, WIDE), jnp.float32),
        pltpu.VMEM((128, WIDE), jnp.float32),
        pltpu.VMEM((B_PER_W,), jnp.float32),
        pltpu.SemaphoreType.DMA,
        pltpu.SemaphoreType.DMA,
    ],
    compiler_params=_params,
)
def _k2(su_hbm, si_hbm, out_hbm, ub0, ib0, ub1, ib1, out_v, sem0, sem1):
    wid = lax.axis_index("s") * 2 + lax.axis_index("c")
    base = wid * B_PER_W
    lane = lax.iota(jnp.int32, 16)
    bufs = [(ub0, ib0, sem0), (ub1, ib1, sem1)]

    def fire(j):
        ub, ib, sem = bufs[j & 1]
        s = pl.ds(base + j * 128, 128)
        return (pltpu.async_copy(su_hbm.at[s, :], ub, sem),
                pltpu.async_copy(si_hbm.at[s, :], ib, sem))

    pending = fire(0)
    for j in range(4):
        nxt = fire(j + 1) if j + 1 < 4 else None
        for h in pending:
            h.wait()
        ubuf, ibuf, _ = bufs[j & 1]

        def body(g, carry, ubuf=ubuf, ibuf=ibuf, j=j):
            row = g * 16 + lane
            acc = jnp.zeros((16,), jnp.float32)
            for c in range(FACTOR):
                col = jnp.full((16,), c, jnp.int32)
                u = plsc.load_gather(ubuf, [row, col])
                v = plsc.load_gather(ibuf, [row, col])
                acc = acc + u * v
            out_v[pl.ds(j * 128 + g * 16, 16)] = acc
            return carry

        lax.fori_loop(0, 8, body, 0)
        pending = nxt

    pltpu.sync_copy(out_v, out_hbm.at[pl.ds(base, B_PER_W)])


def kernel(user, item, embed_user, embed_item):
    eu3 = embed_user.T.reshape(8, 8, NROWS)
    ei3 = embed_item.T.reshape(8, 8, NROWS)
    su, si = _k1(user.astype(jnp.int32), item.astype(jnp.int32), eu3, ei3)
    return _k2(su, si)


# speedup vs baseline: 1.0774x; 1.0276x over previous
import functools

import jax
import jax.numpy as jnp
from jax import lax
from jax.experimental import pallas as pl
from jax.experimental.pallas import tpu as pltpu
from jax.experimental.pallas import tpu_sc as plsc

BATCH = 16384
FACTOR = 64
NW = 32
B_PER_W = BATCH // NW
NROWS = 1000000
NWIN = 1954
WROWS = 512
LAST_WSTART = 999552
STAGE_ROWS = BATCH + 128
LISTCAP = 2048
WIDE = 128

_mesh = plsc.VectorSubcoreMesh(core_axis_name="c", subcore_axis_name="s")
_params = pltpu.CompilerParams(needs_layout_passes=False, use_tc_tiling_on_sc=True)


@functools.partial(
    pl.kernel,
    mesh=_mesh,
    out_type=(
        jax.ShapeDtypeStruct((STAGE_ROWS, WIDE), jnp.float32),
        jax.ShapeDtypeStruct((STAGE_ROWS, WIDE), jnp.float32),
    ),
    scratch_types=[
        pltpu.VMEM((BATCH,), jnp.int32),
        pltpu.VMEM((LISTCAP + 64,), jnp.int32),
        pltpu.VMEM((LISTCAP + 64,), jnp.int32),
        pltpu.VMEM((32,), jnp.int32),
        pltpu.VMEM((32,), jnp.int32),
        pltpu.VMEM((8, 8, WROWS), jnp.float32),
        pltpu.VMEM((8, 8, WROWS), jnp.float32),
        pltpu.VMEM((128, WIDE), jnp.float32),
        pltpu.VMEM((128,), jnp.int32),
        pltpu.SemaphoreType.DMA,
        pltpu.SemaphoreType.DMA,
        pltpu.SemaphoreType.DMA,
    ],
    compiler_params=_params,
)
def _k1(user_hbm, item_hbm, eu_hbm, ei_hbm, su_hbm, si_hbm,
        idx_v, rl, bl, rblk, bblk, blka, blkb, rowbuf, bchunk,
        sema, semb, sems):
    wid = lax.axis_index("s") * 2 + lax.axis_index("c")
    lo_w = (wid * NWIN) // NW
    hi_w = ((wid + 1) * NWIN) // NW
    lane = lax.iota(jnp.int32, 16)
    ci = lane & 7
    cbs = [2 * k + (lane >> 3) for k in range(4)]

    def reset_bchunk():
        for k in range(8):
            bchunk[pl.ds(k * 16, 16)] = BATCH + k * 16 + lane

    reset_bchunk()

    def flush(stage_hbm):
        pltpu.async_copy(rowbuf, stage_hbm.at[bchunk], sems).wait()
        reset_bchunk()

    def wstart(j):
        return pl.multiple_of(jnp.minimum(j * WROWS, LAST_WSTART), 128)

    def blk_slice(tref, j):
        return tref.at[:, :, pl.ds(wstart(j), WROWS)]

    def run_table(idx_hbm, tref, stage_hbm):
        pltpu.sync_copy(idx_hbm, idx_v)

        def scan_body(q, pos):
            ms, pcs = [], []
            for k in range(4):
                r16 = idx_v[pl.ds(q * 64 + k * 16, 16)]
                w = r16 >> 9
                m = (w >= lo_w) & (w < hi_w)
                ms.append((r16, m))
                pcs.append(jnp.sum(m.astype(jnp.int32)))
            for k in range(4):
                r16, m = ms[k]
                plsc.store_compressed(rl.at[pl.ds(pos, 16)], r16, mask=m)
                plsc.store_compressed(
                    bl.at[pl.ds(pos, 16)], q * 64 + k * 16 + lane, mask=m)
                pos = jnp.minimum(pos + pcs[k], LISTCAP)
            return pos

        cnt = lax.fori_loop(0, BATCH // 64, scan_body, 0)
        for k in range(4):
            rl[pl.ds(cnt + k * 16, 16)] = jnp.full((16,), -1, jnp.int32)

        nq = (cnt + 63) >> 6

        def process(j, blkref, hc):
            start = wstart(j)

            def sub(q, hc):
                r16s, ms, pcs = [], [], []
                for k in range(4):
                    r16 = rl[pl.ds(q * 64 + k * 16, 16)]
                    m = (r16 >> 9) == j
                    r16s.append(r16)
                    ms.append(m)
                    pcs.append(jnp.sum(m.astype(jnp.int32)))

                for k in range(4):
                    r16, m, pc = r16s[k], ms[k], pcs[k]

                    def have(hc, r16=r16, m=m, pc=pc, k=k):
                        plsc.store_compressed(rblk.at[pl.ds(0, 16)], r16, mask=m)
                        plsc.store_compressed(
                            bblk.at[pl.ds(0, 16)],
                            bl[pl.ds(q * 64 + k * 16, 16)], mask=m)

                        def per_hit(h, hc):
                            rvec = rblk[pl.ds(h, 16)]
                            bvec = bblk[pl.ds(h, 16)]
                            ri = jnp.full((16,), rvec[0] - start, jnp.int32)
                            for t in range(4):
                                val = plsc.load_gather(blkref, [cbs[t], ci, ri])
                                rowbuf[hc, pl.ds(t * 16, 16)] = val
                            grp = (hc >> 4) * 16
                            off = hc & 15
                            cur = bchunk[pl.ds(grp, 16)]
                            bchunk[pl.ds(grp, 16)] = jnp.where(
                                lane == off,
                                jnp.full((16,), bvec[0], jnp.int32), cur)
                            hc = hc + 1

                            def do_flush(hc):
                                flush(stage_hbm)
                                return 0

                            return lax.cond(hc == 128, do_flush,
                                            lambda hc: hc, hc)

                        return lax.fori_loop(0, pc, per_hit, hc)

                    hc = lax.cond(pc > 0, have, lambda hc: hc, hc)
                return hc

            return lax.fori_loop(0, nq, sub, hc)

        def fire(j, buf, sem):
            pltpu.async_copy(blk_slice(tref, j), buf, sem)

        def wait(j, buf, sem):
            pltpu.make_async_copy(blk_slice(tref, j), buf, sem).wait()

        @pl.when(lo_w < hi_w)
        def _():
            fire(lo_w, blka, sema)

        npairs = (hi_w - lo_w + 1) // 2

        def pair(t, hc):
            j0 = lo_w + 2 * t
            j1 = j0 + 1
            j2 = j0 + 2

            @pl.when(j1 < hi_w)
            def _():
                fire(j1, blkb, semb)

            wait(j0, blka, sema)
            hc = process(j0, blka, hc)

            @pl.when(j2 < hi_w)
            def _():
                fire(j2, blka, sema)

            def do_b(hc):
                wait(j1, blkb, semb)
                return process(j1, blkb, hc)

            return lax.cond(j1 < hi_w, do_b, lambda hc: hc, hc)

        hc = lax.fori_loop(0, npairs, pair, 0)

        @pl.when(hc > 0)
        def _():
            flush(stage_hbm)

    run_table(user_hbm, eu_hbm, su_hbm)
    run_table(item_hbm, ei_hbm, si_hbm)


@functools.partial(
    pl.kernel,
    mesh=_mesh,
    out_type=jax.ShapeDtypeStruct((BATCH,), jnp.float32),
    scratch_types=[
        pltpu.VMEM((128, WIDE), jnp.float32),
        pltpu.VMEM((128, WIDE), jnp.float32),
        pltpu.VMEM((128, WIDE), jnp.float32),
        pltpu.VMEM((128, WIDE), jnp.float32),
        pltpu.VMEM((B_PER_W,), jnp.float32),
        pltpu.SemaphoreType.DMA,
        pltpu.SemaphoreType.DMA,
    ],
    compiler_params=_params,
)
def _k2(su_hbm, si_hbm, out_hbm, ub0, ib0, ub1, ib1, out_v, sem0, sem1):
    wid = lax.axis_index("s") * 2 + lax.axis_index("c")
    base = wid * B_PER_W
    lane = lax.iota(jnp.int32, 16)
    bufs = [(ub0, ib0, sem0), (ub1, ib1, sem1)]

    def fire(j):
        ub, ib, sem = bufs[j & 1]
        s = pl.ds(base + j * 128, 128)
        return (pltpu.async_copy(su_hbm.at[s, :], ub, sem),
                pltpu.async_copy(si_hbm.at[s, :], ib, sem))

    pending = fire(0)
    for j in range(4):
        nxt = fire(j + 1) if j + 1 < 4 else None
        for h in pending:
            h.wait()
        ubuf, ibuf, _ = bufs[j & 1]

        def body(g, carry, ubuf=ubuf, ibuf=ibuf, j=j):
            row = g * 16 + lane
            accs = [jnp.zeros((16,), jnp.float32) for _ in range(4)]
            for c in range(FACTOR):
                col = jnp.full((16,), c, jnp.int32)
                u = plsc.load_gather(ubuf, [row, col])
                v = plsc.load_gather(ibuf, [row, col])
                accs[c & 3] = accs[c & 3] + u * v
            out_v[pl.ds(j * 128 + g * 16, 16)] = (
                (accs[0] + accs[1]) + (accs[2] + accs[3]))
            return carry

        lax.fori_loop(0, 8, body, 0)
        pending = nxt

    pltpu.sync_copy(out_v, out_hbm.at[pl.ds(base, B_PER_W)])


def kernel(user, item, embed_user, embed_item):
    eu3 = embed_user.T.reshape(8, 8, NROWS)
    ei3 = embed_item.T.reshape(8, 8, NROWS)
    su, si = _k1(user.astype(jnp.int32), item.astype(jnp.int32), eu3, ei3)
    return _k2(su, si)
